# baseline (speedup 1.0000x reference)
import functools

import jax
import jax.numpy as jnp
from jax import lax
from jax.experimental import pallas as pl
from jax.experimental.pallas import tpu as pltpu
from jax.experimental.pallas import tpu_sc as plsc

INFLOW = 4
OUTFLOW = 5
WALL = 6
PRESS_POINT = 7

NC = 2
NS = 16
NW = NC * NS
L = 16

CH = 4096
TILE_ROWS = 37504

GS = 6272
GROWS = GS // 128
G_REAL = 100000
N_EXT_C = 1200000
T_PAD = N_EXT_C + 256
HALF = 16 * TILE_ROWS
ROUNDS = 6
RCH = 1024


def _iota16():
    return lax.iota(jnp.int32, L)


def _p1_chunk(uvp_f, ext_idx, ext_nt, out_f, idx_v, nt_v, i0_v, i1_v, i2_v,
              c0_v, c1_v, c2_v, il_v, sem, base, n):
    pltpu.sync_copy(ext_idx.at[pl.ds(base, n)], idx_v.at[pl.ds(0, n)])
    pltpu.sync_copy(ext_nt.at[pl.ds(base, n)], nt_v.at[pl.ds(0, n)])

    @pl.loop(0, n // L)
    def _(i):
        lanes = i * L + _iota16()
        v3 = idx_v[pl.ds(i * L, L)] * 3
        plsc.store_scatter(i0_v, [lanes], v3)
        plsc.store_scatter(i1_v, [lanes], v3 + 1)
        plsc.store_scatter(i2_v, [lanes], v3 + 2)

    @pl.loop(0, n // 512)
    def _(g):
        descs = []
        for jb in range(4):
            o = g * 512 + jb * 128
            for ib, cb in ((i0_v, c0_v), (i1_v, c1_v), (i2_v, c2_v)):
                descs.append(pltpu.async_copy(
                    uvp_f.at[ib.at[pl.ds(o, 128)]],
                    cb.at[pl.ds(o, 128)], sem))
        for d in descs:
            d.wait()

    tail_descs = []
    for jb in range((n % 512) // 128):
        o = (n // 512) * 512 + jb * 128
        for ib, cb in ((i0_v, c0_v), (i1_v, c1_v), (i2_v, c2_v)):
            tail_descs.append(pltpu.async_copy(
                uvp_f.at[ib.at[pl.ds(o, 128)]],
                cb.at[pl.ds(o, 128)], sem))
    for d in tail_descs:
        d.wait()

    @pl.loop(0, n // L)
    def _(i):
        lanes = i * L + _iota16()
        pos = lanes * 3
        press = nt_v[pl.ds(i * L, L)] == PRESS_POINT
        pv = jnp.where(press, 0.0, c2_v[pl.ds(i * L, L)])
        plsc.store_scatter(il_v, [pos], c0_v[pl.ds(i * L, L)])
        plsc.store_scatter(il_v, [pos + 1], c1_v[pl.ds(i * L, L)])
        plsc.store_scatter(il_v, [pos + 2], pv)

    pltpu.sync_copy(il_v.at[pl.ds(0, 3 * n)], out_f.at[pl.ds(3 * base, 3 * n)])


def _phase1(uvp_f, ext_idx, ext_nt, out_f, idx_v, nt_v, i0_v, i1_v, i2_v,
            c0_v, c1_v, c2_v, il_v, sem, w):
    tb = w * TILE_ROWS
    for j in range(9):
        _p1_chunk(uvp_f, ext_idx, ext_nt, out_f, idx_v, nt_v, i0_v, i1_v,
                  i2_v, c0_v, c1_v, c2_v, il_v, sem, tb + j * CH, CH)

    @pl.when(w != NW - 1)
    def _():
        _p1_chunk(uvp_f, ext_idx, ext_nt, out_f, idx_v, nt_v, i0_v, i1_v,
                  i2_v, c0_v, c1_v, c2_v, il_v, sem, tb + 9 * CH, 640)

    @pl.when(w == NW - 1)
    def _():
        _p1_chunk(uvp_f, ext_idx, ext_nt, out_f, idx_v, nt_v, i0_v, i1_v,
                  i2_v, c0_v, c1_v, c2_v, il_v, sem, tb + 9 * CH, 512)


def _dedup_body(gcol, t_out, g1_v, k1_v, gr1_v, t1_v, T_sh, sem):
    s = lax.axis_index("s")
    kbase = s * GS

    pltpu.sync_copy(gcol.at[pl.ds(kbase, GS)], g1_v)

    @pl.loop(0, GS // L)
    def _(u):
        lanes = u * L + _iota16()
        gk = kbase + lanes
        valid = gk < G_REAL
        k16 = jnp.where(valid, gk, -2)
        dum = N_EXT_C + (k16 & 255)
        gsafe = jnp.where(valid, g1_v[pl.ds(u * L, L)], dum)
        plsc.store_scatter(g1_v, [lanes], gsafe)
        plsc.store_scatter(k1_v, [lanes], k16)
        plsc.store_scatter(gr1_v, [lanes], gsafe)

    for _r in range(ROUNDS):
        @pl.loop(0, GROWS)
        def _(jb):
            o = jb * 128
            pltpu.sync_copy(k1_v.at[pl.ds(o, 128)],
                            T_sh.at[gr1_v.at[pl.ds(o, 128)]])
        plsc.subcore_barrier()

        @pl.loop(0, GROWS)
        def _(jb):
            o = jb * 128
            pltpu.async_copy(T_sh.at[gr1_v.at[pl.ds(o, 128)]],
                             t1_v.at[pl.ds(o, 128)], sem).wait()
        if _r < ROUNDS - 1:
            @pl.loop(0, GS // L)
            def _(u):
                lanes = u * L + _iota16()
                k16 = k1_v[pl.ds(u * L, L)]
                t16 = t1_v[pl.ds(u * L, L)]
                gr16 = gr1_v[pl.ds(u * L, L)]
                g16 = g1_v[pl.ds(u * L, L)]
                act = (gr16 < N_EXT_C) & (k16 > t16)
                dum = N_EXT_C + (k16 & 255)
                plsc.store_scatter(gr1_v, [lanes], jnp.where(act, g16, dum))
        plsc.subcore_barrier()

    @pl.loop(0, GROWS)
    def _(jb):
        o = jb * 128
        pltpu.async_copy(T_sh.at[g1_v.at[pl.ds(o, 128)]],
                         t1_v.at[pl.ds(o, 128)], sem).wait()

    pltpu.sync_copy(t1_v, t_out.at[pl.ds(kbase, GS)])


def _dedup_call(gcol):
    mesh = plsc.VectorSubcoreMesh(
        core_axis_name="c", subcore_axis_name="s",
        num_cores=NC, num_subcores=NS)
    i32 = jnp.int32
    return pl.kernel(
        _dedup_body,
        out_type=jax.ShapeDtypeStruct((NS * GS,), i32),
        mesh=mesh,
        compiler_params=pltpu.CompilerParams(
            needs_layout_passes=False, use_tc_tiling_on_sc=False),
        scratch_types=[
            pltpu.VMEM((GS,), i32),
            pltpu.VMEM((GS,), i32),
            pltpu.VMEM((GS,), i32),
            pltpu.VMEM((GS,), i32),
            pltpu.VMEM_SHARED((T_PAD,), i32),
            pltpu.SemaphoreType.DMA,
        ],
    )(gcol)


def _body(uvp_f, y_f, node_type, ext_idx, ext_nt, gcol, n1col, n2col, t_all,
          out_f,
          idx_v, nt_v, i0_v, i1_v, i2_v, c0_v, c1_v, c2_v, il_v,
          g1_v, t1_v, n11_v, n21_v,
          cg_v, cn1_v, cn2_v,
          e1_v, e2_v, eg_v, gt_v, nt1_v,
          e1s_v, e2s_v, egs_v, cgs_v,
          u1s_v, u2s_v, ugs_v, y1s_v, nrs_v,
          sem):
    c = lax.axis_index("c")
    s = lax.axis_index("s")
    w = c * NS + s
    c_is1 = c == 1
    kbase = s * GS

    pltpu.sync_copy(gcol.at[pl.ds(kbase, GS)], g1_v)
    pltpu.sync_copy(n1col.at[pl.ds(kbase, GS)], n11_v)
    pltpu.sync_copy(n2col.at[pl.ds(kbase, GS)], n21_v)
    pltpu.sync_copy(t_all.at[pl.ds(kbase, GS)], t1_v)

    _phase1(uvp_f, ext_idx, ext_nt, out_f, idx_v, nt_v, i0_v, i1_v, i2_v,
            c0_v, c1_v, c2_v, il_v, sem, w)
    plsc.subcore_barrier()

    def compact(u, m):
        lanes = u * L + _iota16()
        gk = kbase + lanes
        t16 = t1_v[pl.ds(u * L, L)]
        g16 = g1_v[pl.ds(u * L, L)]
        own = (g16 < HALF) != c_is1
        sel = (gk == t16) & own & (gk < G_REAL)
        sel_i = sel.astype(jnp.int32)
        pos = m + plsc.cumsum(sel_i) - 1
        plsc.store_scatter(cg_v, [pos], g16, mask=sel)
        plsc.store_scatter(cn1_v, [pos], n11_v[pl.ds(u * L, L)], mask=sel)
        plsc.store_scatter(cn2_v, [pos], n21_v[pl.ds(u * L, L)], mask=sel)
        return m + jnp.sum(sel_i)

    m = pl.loop(0, GS // L, init_carry=jnp.int32(0))(compact)

    @pl.when(m > 0)
    def _():
        fm = jnp.full((L,), m - 1, jnp.int32)
        lastg = plsc.load_gather(cg_v, [fm])
        lastn1 = plsc.load_gather(cn1_v, [fm])
        lastn2 = plsc.load_gather(cn2_v, [fm])
        mpad = ((m + 127) // 128) * 128

        @pl.loop(m // L, mpad // L)
        def _(u):
            pos = u * L + _iota16()
            mask = pos >= m
            plsc.store_scatter(cg_v, [pos], lastg, mask=mask)
            plsc.store_scatter(cn1_v, [pos], lastn1, mask=mask)
            plsc.store_scatter(cn2_v, [pos], lastn2, mask=mask)

        @pl.loop(0, mpad // 128)
        def _(jb):
            o = jb * 128
            d1 = pltpu.async_copy(ext_nt.at[cg_v.at[pl.ds(o, 128)]], gt_v, sem)
            d2 = pltpu.async_copy(ext_idx.at[cg_v.at[pl.ds(o, 128)]], eg_v, sem)
            d3 = pltpu.async_copy(ext_idx.at[cn1_v.at[pl.ds(o, 128)]],
                                  e1_v, sem)
            d4 = pltpu.async_copy(ext_idx.at[cn2_v.at[pl.ds(o, 128)]],
                                  e2_v, sem)
            d1.wait(); d2.wait(); d3.wait(); d4.wait()

            @pl.loop(0, 128 // L)
            def _(u):
                lanes = u * L + _iota16()
                for ref, base_v in ((e1s_v, e1_v), (e2s_v, e2_v),
                                    (egs_v, eg_v)):
                    v3 = base_v[pl.ds(u * L, L)] * 3
                    plsc.store_scatter(ref, [lanes], v3)
                    plsc.store_scatter(ref, [lanes + 128], v3 + 1)
                    plsc.store_scatter(ref, [lanes + 256], v3 + 2)
                g3 = cg_v[pl.ds(o + u * L, L)] * 3
                plsc.store_scatter(cgs_v, [lanes], g3)
                plsc.store_scatter(cgs_v, [lanes + 128], g3 + 1)
                plsc.store_scatter(cgs_v, [lanes + 256], g3 + 2)

            descs = [pltpu.async_copy(node_type.at[e1_v], nt1_v, sem)]
            for cc in range(3):
                oo = cc * 128
                descs.append(pltpu.async_copy(
                    uvp_f.at[e1s_v.at[pl.ds(oo, 128)]],
                    u1s_v.at[pl.ds(oo, 128)], sem))
                descs.append(pltpu.async_copy(
                    uvp_f.at[e2s_v.at[pl.ds(oo, 128)]],
                    u2s_v.at[pl.ds(oo, 128)], sem))
                descs.append(pltpu.async_copy(
                    uvp_f.at[egs_v.at[pl.ds(oo, 128)]],
                    ugs_v.at[pl.ds(oo, 128)], sem))
                if cc < 2:
                    descs.append(pltpu.async_copy(
                        y_f.at[e1s_v.at[pl.ds(oo, 128)]],
                        y1s_v.at[pl.ds(oo, 128)], sem))
            for d in descs:
                d.wait()

            @pl.loop(0, 128 // L)
            def _(u):
                ol = u * L
                gt16 = gt_v[pl.ds(ol, L)]
                nt116 = nt1_v[pl.ds(ol, L)]
                uvN = gt16 == OUTFLOW
                pN = (gt16 == WALL) | (gt16 == INFLOW)
                bc1 = (nt116 == INFLOW) | (nt116 == WALL)
                out1 = nt116 == OUTFLOW
                for cc in (0, 1):
                    oo = cc * 128 + ol
                    u1c = u1s_v[pl.ds(oo, L)]
                    u2c = u2s_v[pl.ds(oo, L)]
                    ugc = ugs_v[pl.ds(oo, L)]
                    y1c = y1s_v[pl.ds(oo, L)]
                    d1c = jnp.where(bc1, y1c, u1c)
                    nrs_v[pl.ds(oo, L)] = jnp.where(
                        pN, 2.0 * d1c - u2c, jnp.where(uvN, u2c, ugc))
                oo = 256 + ol
                d1p = jnp.where(out1, 0.0, u1s_v[pl.ds(oo, L)])
                u2p = u2s_v[pl.ds(oo, L)]
                newp = jnp.where(uvN, 2.0 * d1p - u2p,
                                 jnp.where(pN, u2p, ugs_v[pl.ds(oo, L)]))
                nrs_v[pl.ds(oo, L)] = jnp.where(gt16 == PRESS_POINT, 0.0, newp)

            for cc in range(3):
                oo = cc * 128
                pltpu.sync_copy(nrs_v.at[pl.ds(oo, 128)],
                                out_f.at[cgs_v.at[pl.ds(oo, 128)]])


def _sc_call(uvp_f, y_f, node_type, extend_index, ext_node_type,
             gcol, n1col, n2col, t_all):
    n_ext = extend_index.shape[0]
    mesh = plsc.VectorSubcoreMesh(
        core_axis_name="c", subcore_axis_name="s",
        num_cores=NC, num_subcores=NS)
    f32 = jnp.float32
    i32 = jnp.int32
    return pl.kernel(
        _body,
        out_type=jax.ShapeDtypeStruct((n_ext * 3,), f32),
        mesh=mesh,
        compiler_params=pltpu.CompilerParams(
            needs_layout_passes=False, use_tc_tiling_on_sc=False),
        scratch_types=[
            pltpu.VMEM((CH,), i32),
            pltpu.VMEM((CH,), i32),
            pltpu.VMEM((CH,), i32),
            pltpu.VMEM((CH,), i32),
            pltpu.VMEM((CH,), i32),
            pltpu.VMEM((CH,), f32),
            pltpu.VMEM((CH,), f32),
            pltpu.VMEM((CH,), f32),
            pltpu.VMEM((CH * 3,), f32),
            pltpu.VMEM((GS,), i32),
            pltpu.VMEM((GS,), i32),
            pltpu.VMEM((GS,), i32),
            pltpu.VMEM((GS,), i32),
            pltpu.VMEM((GS,), i32),
            pltpu.VMEM((GS,), i32),
            pltpu.VMEM((GS,), i32),
            pltpu.VMEM((128,), i32),
            pltpu.VMEM((128,), i32),
            pltpu.VMEM((128,), i32),
            pltpu.VMEM((128,), i32),
            pltpu.VMEM((128,), i32),
            pltpu.VMEM((384,), i32),
            pltpu.VMEM((384,), i32),
            pltpu.VMEM((384,), i32),
            pltpu.VMEM((384,), i32),
            pltpu.VMEM((384,), f32),
            pltpu.VMEM((384,), f32),
            pltpu.VMEM((384,), f32),
            pltpu.VMEM((256,), f32),
            pltpu.VMEM((384,), f32),
            pltpu.SemaphoreType.DMA,
        ],
    )(uvp_f, y_f, node_type, extend_index, ext_node_type, gcol, n1col,
      n2col, t_all)


def kernel(uvp, y, node_type, extend_index, ext_node_type,
           boundary_ghost_stencil_index):
    pad = NS * GS - G_REAL
    gcol = jnp.pad(boundary_ghost_stencil_index[:, 0], (0, pad)).astype(
        jnp.int32)
    n1col = jnp.pad(boundary_ghost_stencil_index[:, 1], (0, pad)).astype(
        jnp.int32)
    n2col = jnp.pad(boundary_ghost_stencil_index[:, 2], (0, pad)).astype(
        jnp.int32)
    t_all = _dedup_call(gcol)
    out_f = _sc_call(uvp.reshape(-1), y.reshape(-1), node_type,
                     extend_index, ext_node_type, gcol, n1col, n2col, t_all)
    return out_f.reshape(extend_index.shape[0], 3)


# speedup vs baseline: 9.9582x; 9.9582x over previous
import functools

import jax
import jax.numpy as jnp
from jax import lax
from jax.experimental import pallas as pl
from jax.experimental.pallas import tpu as pltpu
from jax.experimental.pallas import tpu_sc as plsc

INFLOW = 4
OUTFLOW = 5
WALL = 6
PRESS_POINT = 7

NC = 2
NS = 16
NW = NC * NS
L = 16

CH = 4096
TILE_ROWS = 37504

GS = 6272
GROWS = GS // 128
G_REAL = 100000
N_EXT_C = 1200000
T_PAD = N_EXT_C + 256
HALF = 16 * TILE_ROWS
ROUNDS = 6
RCH = 1024


def _iota16():
    return lax.iota(jnp.int32, L)


def _p1_chunk(ux, uy, up, ext_idx, ext_nt, ox, oy, op, idx_v, nt_v,
              c0_v, c1_v, c2_v, sem, base, n):
    pltpu.sync_copy(ext_idx.at[pl.ds(base, n)], idx_v.at[pl.ds(0, n)])
    pltpu.sync_copy(ext_nt.at[pl.ds(base, n)], nt_v.at[pl.ds(0, n)])

    def fire(o):
        return [pltpu.async_copy(tab.at[idx_v.at[pl.ds(o, 128)]],
                                 cb.at[pl.ds(o, 128)], sem)
                for tab, cb in ((ux, c0_v), (uy, c1_v), (up, c2_v))]

    @pl.loop(0, n // 512)
    def _(g):
        descs = []
        for jb in range(4):
            descs += fire(g * 512 + jb * 128)
        for d in descs:
            d.wait()

    tail_descs = []
    for jb in range((n % 512) // 128):
        tail_descs += fire((n // 512) * 512 + jb * 128)
    for d in tail_descs:
        d.wait()

    zero = jnp.zeros((L,), jnp.float32)

    @pl.loop(0, n // L)
    def _(i):
        press = nt_v[pl.ds(i * L, L)] == PRESS_POINT
        plsc.store_scatter(c2_v, [i * L + _iota16()], zero, mask=press)

    pltpu.sync_copy(c0_v.at[pl.ds(0, n)], ox.at[pl.ds(base, n)])
    pltpu.sync_copy(c1_v.at[pl.ds(0, n)], oy.at[pl.ds(base, n)])
    pltpu.sync_copy(c2_v.at[pl.ds(0, n)], op.at[pl.ds(base, n)])


def _phase1(ux, uy, up, ext_idx, ext_nt, ox, oy, op, idx_v, nt_v,
            c0_v, c1_v, c2_v, sem, w):
    tb = w * TILE_ROWS
    for j in range(9):
        _p1_chunk(ux, uy, up, ext_idx, ext_nt, ox, oy, op, idx_v, nt_v,
                  c0_v, c1_v, c2_v, sem, tb + j * CH, CH)

    @pl.when(w != NW - 1)
    def _():
        _p1_chunk(ux, uy, up, ext_idx, ext_nt, ox, oy, op, idx_v, nt_v,
                  c0_v, c1_v, c2_v, sem, tb + 9 * CH, 640)

    @pl.when(w == NW - 1)
    def _():
        _p1_chunk(ux, uy, up, ext_idx, ext_nt, ox, oy, op, idx_v, nt_v,
                  c0_v, c1_v, c2_v, sem, tb + 9 * CH, 512)


def _dedup_body(gcol, t_out, g1_v, k1_v, gr1_v, t1_v, T_sh, sem):
    s = lax.axis_index("s")
    kbase = s * GS

    pltpu.sync_copy(gcol.at[pl.ds(kbase, GS)], g1_v)

    @pl.loop(0, GS // L)
    def _(u):
        lanes = u * L + _iota16()
        gk = kbase + lanes
        valid = gk < G_REAL
        k16 = jnp.where(valid, gk, -2)
        dum = N_EXT_C + (k16 & 255)
        gsafe = jnp.where(valid, g1_v[pl.ds(u * L, L)], dum)
        plsc.store_scatter(g1_v, [lanes], gsafe)
        plsc.store_scatter(k1_v, [lanes], k16)
        plsc.store_scatter(gr1_v, [lanes], gsafe)

    for _r in range(ROUNDS):
        @pl.loop(0, GROWS)
        def _(jb):
            o = jb * 128
            pltpu.sync_copy(k1_v.at[pl.ds(o, 128)],
                            T_sh.at[gr1_v.at[pl.ds(o, 128)]])
        plsc.subcore_barrier()

        @pl.loop(0, GROWS)
        def _(jb):
            o = jb * 128
            pltpu.async_copy(T_sh.at[gr1_v.at[pl.ds(o, 128)]],
                             t1_v.at[pl.ds(o, 128)], sem).wait()
        if _r < ROUNDS - 1:
            @pl.loop(0, GS // L)
            def _(u):
                lanes = u * L + _iota16()
                k16 = k1_v[pl.ds(u * L, L)]
                t16 = t1_v[pl.ds(u * L, L)]
                gr16 = gr1_v[pl.ds(u * L, L)]
                g16 = g1_v[pl.ds(u * L, L)]
                act = (gr16 < N_EXT_C) & (k16 > t16)
                dum = N_EXT_C + (k16 & 255)
                plsc.store_scatter(gr1_v, [lanes], jnp.where(act, g16, dum))
        plsc.subcore_barrier()

    @pl.loop(0, GROWS)
    def _(jb):
        o = jb * 128
        pltpu.async_copy(T_sh.at[g1_v.at[pl.ds(o, 128)]],
                         t1_v.at[pl.ds(o, 128)], sem).wait()

    pltpu.sync_copy(t1_v, t_out.at[pl.ds(kbase, GS)])


def _dedup_call(gcol):
    mesh = plsc.VectorSubcoreMesh(
        core_axis_name="c", subcore_axis_name="s",
        num_cores=NC, num_subcores=NS)
    i32 = jnp.int32
    return pl.kernel(
        _dedup_body,
        out_type=jax.ShapeDtypeStruct((NS * GS,), i32),
        mesh=mesh,
        compiler_params=pltpu.CompilerParams(
            needs_layout_passes=False, use_tc_tiling_on_sc=False),
        scratch_types=[
            pltpu.VMEM((GS,), i32),
            pltpu.VMEM((GS,), i32),
            pltpu.VMEM((GS,), i32),
            pltpu.VMEM((GS,), i32),
            pltpu.VMEM_SHARED((T_PAD,), i32),
            pltpu.SemaphoreType.DMA,
        ],
    )(gcol)


def _body(ux, uy, up, yx, yy, node_type, ext_idx, ext_nt, gcol, n1col, n2col,
          t_all, ox, oy, op,
          idx_v, nt_v, c0_v, c1_v, c2_v,
          g1_v, t1_v, n11_v, n21_v,
          cg_v, cn1_v, cn2_v,
          e1_v, e2_v, eg_v, gt_v, nt1_v,
          u1s_v, u2s_v, ugs_v, y1s_v, nrs_v,
          sem):
    c = lax.axis_index("c")
    s = lax.axis_index("s")
    w = c * NS + s
    c_is1 = c == 1
    kbase = s * GS

    pltpu.sync_copy(gcol.at[pl.ds(kbase, GS)], g1_v)
    pltpu.sync_copy(n1col.at[pl.ds(kbase, GS)], n11_v)
    pltpu.sync_copy(n2col.at[pl.ds(kbase, GS)], n21_v)
    pltpu.sync_copy(t_all.at[pl.ds(kbase, GS)], t1_v)

    _phase1(ux, uy, up, ext_idx, ext_nt, ox, oy, op, idx_v, nt_v,
            c0_v, c1_v, c2_v, sem, w)
    plsc.subcore_barrier()

    def compact(u, m):
        lanes = u * L + _iota16()
        gk = kbase + lanes
        t16 = t1_v[pl.ds(u * L, L)]
        g16 = g1_v[pl.ds(u * L, L)]
        own = (g16 < HALF) != c_is1
        sel = (gk == t16) & own & (gk < G_REAL)
        sel_i = sel.astype(jnp.int32)
        pos = m + plsc.cumsum(sel_i) - 1
        plsc.store_scatter(cg_v, [pos], g16, mask=sel)
        plsc.store_scatter(cn1_v, [pos], n11_v[pl.ds(u * L, L)], mask=sel)
        plsc.store_scatter(cn2_v, [pos], n21_v[pl.ds(u * L, L)], mask=sel)
        return m + jnp.sum(sel_i)

    m = pl.loop(0, GS // L, init_carry=jnp.int32(0))(compact)

    @pl.when(m > 0)
    def _():
        fm = jnp.full((L,), m - 1, jnp.int32)
        lastg = plsc.load_gather(cg_v, [fm])
        lastn1 = plsc.load_gather(cn1_v, [fm])
        lastn2 = plsc.load_gather(cn2_v, [fm])
        mpad = ((m + 127) // 128) * 128

        @pl.loop(m // L, mpad // L)
        def _(u):
            pos = u * L + _iota16()
            mask = pos >= m
            plsc.store_scatter(cg_v, [pos], lastg, mask=mask)
            plsc.store_scatter(cn1_v, [pos], lastn1, mask=mask)
            plsc.store_scatter(cn2_v, [pos], lastn2, mask=mask)

        @pl.loop(0, mpad // 128)
        def _(jb):
            o = jb * 128
            d1 = pltpu.async_copy(ext_nt.at[cg_v.at[pl.ds(o, 128)]], gt_v, sem)
            d2 = pltpu.async_copy(ext_idx.at[cg_v.at[pl.ds(o, 128)]], eg_v, sem)
            d3 = pltpu.async_copy(ext_idx.at[cn1_v.at[pl.ds(o, 128)]],
                                  e1_v, sem)
            d4 = pltpu.async_copy(ext_idx.at[cn2_v.at[pl.ds(o, 128)]],
                                  e2_v, sem)
            d1.wait(); d2.wait(); d3.wait(); d4.wait()

            descs = [pltpu.async_copy(node_type.at[e1_v], nt1_v, sem)]
            for cc, tab in ((0, ux), (1, uy), (2, up)):
                oo = cc * 128
                descs.append(pltpu.async_copy(
                    tab.at[e1_v], u1s_v.at[pl.ds(oo, 128)], sem))
                descs.append(pltpu.async_copy(
                    tab.at[e2_v], u2s_v.at[pl.ds(oo, 128)], sem))
                descs.append(pltpu.async_copy(
                    tab.at[eg_v], ugs_v.at[pl.ds(oo, 128)], sem))
            descs.append(pltpu.async_copy(
                yx.at[e1_v], y1s_v.at[pl.ds(0, 128)], sem))
            descs.append(pltpu.async_copy(
                yy.at[e1_v], y1s_v.at[pl.ds(128, 128)], sem))
            for d in descs:
                d.wait()

            @pl.loop(0, 128 // L)
            def _(u):
                ol = u * L
                gt16 = gt_v[pl.ds(ol, L)]
                nt116 = nt1_v[pl.ds(ol, L)]
                uvN = gt16 == OUTFLOW
                pN = (gt16 == WALL) | (gt16 == INFLOW)
                bc1 = (nt116 == INFLOW) | (nt116 == WALL)
                out1 = nt116 == OUTFLOW
                for cc in (0, 1):
                    oo = cc * 128 + ol
                    u1c = u1s_v[pl.ds(oo, L)]
                    u2c = u2s_v[pl.ds(oo, L)]
                    ugc = ugs_v[pl.ds(oo, L)]
                    y1c = y1s_v[pl.ds(oo, L)]
                    d1c = jnp.where(bc1, y1c, u1c)
                    nrs_v[pl.ds(oo, L)] = jnp.where(
                        pN, 2.0 * d1c - u2c, jnp.where(uvN, u2c, ugc))
                oo = 256 + ol
                d1p = jnp.where(out1, 0.0, u1s_v[pl.ds(oo, L)])
                u2p = u2s_v[pl.ds(oo, L)]
                newp = jnp.where(uvN, 2.0 * d1p - u2p,
                                 jnp.where(pN, u2p, ugs_v[pl.ds(oo, L)]))
                nrs_v[pl.ds(oo, L)] = jnp.where(gt16 == PRESS_POINT, 0.0, newp)

            for cc, ob in ((0, ox), (1, oy), (2, op)):
                oo = cc * 128
                pltpu.sync_copy(nrs_v.at[pl.ds(oo, 128)],
                                ob.at[cg_v.at[pl.ds(o, 128)]])


def _sc_call(ux, uy, up, yx, yy, node_type, extend_index, ext_node_type,
             gcol, n1col, n2col, t_all):
    n_ext = extend_index.shape[0]
    mesh = plsc.VectorSubcoreMesh(
        core_axis_name="c", subcore_axis_name="s",
        num_cores=NC, num_subcores=NS)
    f32 = jnp.float32
    i32 = jnp.int32
    oshape = jax.ShapeDtypeStruct((n_ext,), f32)
    return pl.kernel(
        _body,
        out_type=(oshape, oshape, oshape),
        mesh=mesh,
        compiler_params=pltpu.CompilerParams(
            needs_layout_passes=False, use_tc_tiling_on_sc=False),
        scratch_types=[
            pltpu.VMEM((CH,), i32),
            pltpu.VMEM((CH,), i32),
            pltpu.VMEM((CH,), f32),
            pltpu.VMEM((CH,), f32),
            pltpu.VMEM((CH,), f32),
            pltpu.VMEM((GS,), i32),
            pltpu.VMEM((GS,), i32),
            pltpu.VMEM((GS,), i32),
            pltpu.VMEM((GS,), i32),
            pltpu.VMEM((GS,), i32),
            pltpu.VMEM((GS,), i32),
            pltpu.VMEM((GS,), i32),
            pltpu.VMEM((128,), i32),
            pltpu.VMEM((128,), i32),
            pltpu.VMEM((128,), i32),
            pltpu.VMEM((128,), i32),
            pltpu.VMEM((128,), i32),
            pltpu.VMEM((384,), f32),
            pltpu.VMEM((384,), f32),
            pltpu.VMEM((384,), f32),
            pltpu.VMEM((256,), f32),
            pltpu.VMEM((384,), f32),
            pltpu.SemaphoreType.DMA,
        ],
    )(ux, uy, up, yx, yy, node_type, extend_index, ext_node_type,
      gcol, n1col, n2col, t_all)


def kernel(uvp, y, node_type, extend_index, ext_node_type,
           boundary_ghost_stencil_index):
    pad = NS * GS - G_REAL
    gcol = jnp.pad(boundary_ghost_stencil_index[:, 0], (0, pad)).astype(
        jnp.int32)
    n1col = jnp.pad(boundary_ghost_stencil_index[:, 1], (0, pad)).astype(
        jnp.int32)
    n2col = jnp.pad(boundary_ghost_stencil_index[:, 2], (0, pad)).astype(
        jnp.int32)
    t_all = _dedup_call(gcol)
    ox, oy, op = _sc_call(
        uvp[:, 0], uvp[:, 1], uvp[:, 2], y[:, 0], y[:, 1],
        node_type.astype(jnp.int32), extend_index.astype(jnp.int32),
        ext_node_type.astype(jnp.int32), gcol, n1col, n2col, t_all)
    return jnp.stack([ox, oy, op], axis=1)
